# Initial kernel scaffold; baseline (speedup 1.0000x reference)
import jax
import jax.numpy as jnp
from jax.experimental import pallas as pl


def kernel(images, palettes, temperature):
    raise NotImplementedError("write your pallas kernel here")


# speedup vs baseline: 1.0183x; 1.0183x over previous
import jax
import jax.numpy as jnp
from jax.experimental import pallas as pl
from jax.experimental.pallas import tpu as pltpu


def _palette_quant_body(x_ref, pal_ref, b_ref, o_ref):
    x = x_ref[0]
    pal = pal_ref[0]
    b = b_ref[0]

    x0 = x[0:1, :]
    x1 = x[1:2, :]
    x2 = x[2:3, :]
    p0 = pal[:, 0:1]
    p1 = pal[:, 1:2]
    p2 = pal[:, 2:3]

    t = b + p0 * x0 + p1 * x1 + p2 * x2
    e = jnp.exp(t)

    s = jnp.sum(e, axis=0, keepdims=True)
    n0 = jnp.sum(e * p0, axis=0, keepdims=True)
    n1 = jnp.sum(e * p1, axis=0, keepdims=True)
    n2 = jnp.sum(e * p2, axis=0, keepdims=True)

    inv = 1.0 / s
    o_ref[0] = jnp.concatenate([n0 * inv, n1 * inv, n2 * inv], axis=0)


def kernel(images, palettes, temperature):
    B, H, W, C = images.shape
    K = palettes.shape[1]
    HW = H * W
    N = 2048
    grid = (B, HW // N)

    scale = 2.0 / temperature
    xp = images.reshape(B, HW, C).transpose(0, 2, 1) * scale
    bias = (-jnp.sum(palettes * palettes, axis=-1) / temperature)[..., None]

    out_planar = pl.pallas_call(
        _palette_quant_body,
        grid=grid,
        in_specs=[
            pl.BlockSpec((1, C, N), lambda i, j: (i, 0, j)),
            pl.BlockSpec((1, K, C), lambda i, j: (i, 0, 0)),
            pl.BlockSpec((1, K, 1), lambda i, j: (i, 0, 0)),
        ],
        out_specs=pl.BlockSpec((1, C, N), lambda i, j: (i, 0, j)),
        out_shape=jax.ShapeDtypeStruct((B, C, HW), jnp.float32),
    )(xp, palettes, bias)

    return out_planar.transpose(0, 2, 1).reshape(B, H, W, C)
